# baseline (speedup 1.0000x reference)
import functools

import jax
import jax.numpy as jnp
from jax import lax
from jax.experimental import pallas as pl
from jax.experimental.pallas import tpu as pltpu
from jax.experimental.pallas import tpu_sc as plsc

NU = 50000
NPAD = 50688
NP = 8
NPC = 4
R = 6336
RG = 6336
ACC_ROWS = 6352
STRIPE = 396
CHUNK = 128
ZR = 36

E_UU = 300000
E_UC = 150000
EP_UU = 301056
EP_UC = 150528
CAP_UU = 9728
CAP_UC = 5120
HID = 128
DU0 = 144
DC0 = 64


def _mesh():
    return plsc.VectorSubcoreMesh(core_axis_name="c", subcore_axis_name="s")


def _make_compact(eperw, cap, be, plo):
    nchunks = eperw // be

    @functools.partial(
        pl.kernel,
        out_type=(
            jax.ShapeDtypeStruct((NPC, 32, cap), jnp.int32),
            jax.ShapeDtypeStruct((NPC, 32, cap), jnp.int32),
            jax.ShapeDtypeStruct((32, 16), jnp.int32),
        ),
        mesh=_mesh(),
        compiler_params=pltpu.CompilerParams(use_tc_tiling_on_sc=False, needs_layout_passes=False),
        scratch_types=[
            pltpu.VMEM((be,), jnp.int32),
            pltpu.VMEM((be,), jnp.int32),
            pltpu.VMEM((cap,), jnp.int32),
            pltpu.VMEM((cap,), jnp.int32),
            pltpu.VMEM((cap,), jnp.int32),
            pltpu.VMEM((cap,), jnp.int32),
            pltpu.VMEM((cap,), jnp.int32),
            pltpu.VMEM((cap,), jnp.int32),
            pltpu.VMEM((cap,), jnp.int32),
            pltpu.VMEM((cap,), jnp.int32),
            pltpu.VMEM((16,), jnp.int32),
        ],
    )
    def k(src_h, dst_h, srcs_o, dsts_o, cnt_o,
          es_v, ed_v, sb0, sb1, sb2, sb3, db0, db1, db2, db3, cvec):
        sb = (sb0, sb1, sb2, sb3)
        db = (db0, db1, db2, db3)
        w = lax.axis_index("c") * 16 + lax.axis_index("s")
        base = w * eperw
        lane = lax.iota(jnp.int32, 16)
        offs = (jnp.int32(0),) * NPC
        for ci in range(nchunks):
            pltpu.sync_copy(src_h.at[pl.ds(base + ci * be, be)], es_v)
            pltpu.sync_copy(dst_h.at[pl.ds(base + ci * be, be)], ed_v)

            def body(g, offs):
                sv = es_v[pl.ds(g * 16, 16)]
                dv = ed_v[pl.ds(g * 16, 16)]
                new = []
                for i in range(NPC):
                    lo = (plo + i) * R
                    m = (dv >= lo) & (dv < lo + R)
                    inc = plsc.cumsum(m.astype(jnp.int32))
                    pos = offs[i] + inc - 1
                    plsc.store_scatter(sb[i], [pos], sv, mask=m)
                    plsc.store_scatter(db[i], [pos], dv - lo, mask=m)
                    new.append(offs[i] + jnp.sum(m.astype(jnp.int32)))
                return tuple(new)

            offs = lax.fori_loop(0, be // 16, body, offs)
        zs = jnp.zeros((16,), jnp.int32)
        gs = jnp.full((16,), RG, jnp.int32)
        for i in range(NPC):
            for j in range(256 // 16):
                sb[i][pl.ds(offs[i] + j * 16, 16)] = zs
                db[i][pl.ds(offs[i] + j * 16, 16)] = gs
        cv = jnp.zeros((16,), jnp.int32)
        for i in range(NPC):
            cv = jnp.where(lane == (plo + i), offs[i], cv)
        cvec[...] = cv
        pltpu.sync_copy(cvec, cnt_o.at[w])
        for i in range(NPC):
            pltpu.sync_copy(sb[i], srcs_o.at[i, w])
            pltpu.sync_copy(db[i], dsts_o.at[i, w])

    return k


def _make_segsum(d, cap):
    @functools.partial(
        pl.kernel,
        out_type=jax.ShapeDtypeStruct((NPAD, d), jnp.float32),
        mesh=_mesh(),
        compiler_params=pltpu.CompilerParams(use_tc_tiling_on_sc=False, needs_layout_passes=False),
        scratch_types=[
            pltpu.VMEM_SHARED((ACC_ROWS, d), jnp.float32),
            pltpu.VMEM((cap,), jnp.int32),
            pltpu.VMEM((cap,), jnp.int32),
            pltpu.VMEM((2 * CHUNK, d), jnp.float32),
            pltpu.VMEM((ZR, d), jnp.float32),
            pltpu.VMEM((16,), jnp.int32),
            pltpu.SemaphoreType.DMA,
        ],
    )
    def k(table, srcs, dsts, cnts, agg,
          acc_sh, src_v, dst_v, rows_v, zbuf, cvec, sem):
        c = lax.axis_index("c")
        s = lax.axis_index("s")
        lane = lax.iota(jnp.int32, 16)
        zv = jnp.zeros((16,), jnp.float32)

        def zb(r, carry):
            for j in range(d // 16):
                zbuf[r, pl.ds(j * 16, 16)] = zv
            return carry

        lax.fori_loop(0, ZR, zb, 0)
        for k2 in range(NPC):
            p = NPC * c + k2

            def zrow(t, carry):
                pltpu.sync_copy(
                    zbuf, acc_sh.at[pl.ds(s * STRIPE + t * ZR, ZR)])
                return carry

            lax.fori_loop(0, STRIPE // ZR, zrow, 0)
            plsc.subcore_barrier()
            for sk in range(2):
                w = sk * 16 + s
                pltpu.sync_copy(cnts.at[w], cvec)
                cnt = jnp.sum(jnp.where(lane == p, cvec[...], 0))
                nbig = (cnt + (2 * CHUNK - 1)) >> 8

                pltpu.sync_copy(srcs.at[p, w], src_v)
                pltpu.sync_copy(dsts.at[p, w], dst_v)

                def chunk(j, carry):
                    pltpu.async_copy(
                        table.at[src_v.at[pl.ds(2 * CHUNK * j, 2 * CHUNK)]],
                        rows_v, sem).wait()
                    pltpu.sync_copy(
                        rows_v,
                        acc_sh.at[dst_v.at[pl.ds(2 * CHUNK * j, 2 * CHUNK)]],
                        add=True)
                    return carry

                lax.fori_loop(0, nbig, chunk, 0)
            plsc.subcore_barrier()
            pltpu.sync_copy(
                acc_sh.at[pl.ds(s * STRIPE, STRIPE)],
                agg.at[pl.ds(p * R + s * STRIPE, STRIPE)])
            plsc.subcore_barrier()

    return k


def _make_hist(eperw, be):
    nchunks = eperw // be
    hsz = NPAD + 16

    @functools.partial(
        pl.kernel,
        out_type=jax.ShapeDtypeStruct((32, NPAD), jnp.float32),
        mesh=_mesh(),
        compiler_params=pltpu.CompilerParams(use_tc_tiling_on_sc=False, needs_layout_passes=False),
        scratch_types=[
            pltpu.VMEM((be,), jnp.int32),
            pltpu.VMEM((hsz,), jnp.float32),
        ],
    )
    def k(dst_h, out, ed_v, hist_v):
        w = lax.axis_index("c") * 16 + lax.axis_index("s")
        base = w * eperw
        zv = jnp.zeros((16,), jnp.float32)
        ov = jnp.ones((16,), jnp.float32)

        def zb(r, carry):
            hist_v[pl.ds(r * 16, 16)] = zv
            return carry

        lax.fori_loop(0, hsz // 16, zb, 0)
        for ci in range(nchunks):
            pltpu.sync_copy(dst_h.at[pl.ds(base + ci * be, be)], ed_v)

            def body(g, carry):
                dv = ed_v[pl.ds(g * 16, 16)]
                plsc.addupdate_scatter(hist_v, [dv], ov)
                return carry

            lax.fori_loop(0, be // 16, body, 0)
        pltpu.sync_copy(hist_v.at[pl.ds(0, NPAD)], out.at[w])

    return k


def _make_gather(d, b, perw):
    nch = perw // CHUNK

    @functools.partial(
        pl.kernel,
        out_type=jax.ShapeDtypeStruct((b, d), jnp.float32),
        mesh=_mesh(),
        compiler_params=pltpu.CompilerParams(use_tc_tiling_on_sc=False, needs_layout_passes=False),
        scratch_types=[
            pltpu.VMEM((perw,), jnp.int32),
            pltpu.VMEM((CHUNK, d), jnp.float32),
            pltpu.SemaphoreType.DMA,
        ],
    )
    def k(table, idx_h, out, idx_v, rows_v, sem):
        w = lax.axis_index("c") * 16 + lax.axis_index("s")
        pltpu.sync_copy(idx_h.at[pl.ds(w * perw, perw)], idx_v)
        for j in range(nch):
            pltpu.async_copy(
                table.at[idx_v.at[pl.ds(j * CHUNK, CHUNK)]],
                rows_v, sem).wait()
            pltpu.sync_copy(rows_v, out.at[pl.ds(w * perw + j * CHUNK, CHUNK)])

    return k


_BLK = 128
_NBLK = NPAD // _BLK


def _row_spec(d):
    return pl.BlockSpec((_BLK, d), lambda i: (i, 0))


def _full_spec(shape):
    return pl.BlockSpec(shape, lambda i: tuple(0 for _ in shape))


def _col(v):
    i = lax.broadcasted_iota(jnp.int32, (_BLK, _BLK), 0)
    j = lax.broadcasted_iota(jnp.int32, (_BLK, _BLK), 1)
    diag = jnp.where(i == j, jnp.broadcast_to(v, (_BLK, _BLK)), 0.0)
    return jnp.sum(diag, axis=1, keepdims=True)


def _scales_tc(huu, hcu, huc):
    def body(huu_r, hcu_r, huc_r, dinv_r, iccu_r, icuc_r):
        dg = _col(jnp.sum(huu_r[...], axis=0, keepdims=True))
        cc = _col(jnp.sum(hcu_r[...], axis=0, keepdims=True))
        cu = _col(jnp.sum(huc_r[...], axis=0, keepdims=True))
        dinv_r[...] = jnp.where(dg > 0, lax.rsqrt(jnp.maximum(dg, 1.0)), 0.0)
        iccu_r[...] = 1.0 / jnp.maximum(cc, 1.0)
        icuc_r[...] = 1.0 / jnp.maximum(cu, 1.0)

    sh = jax.ShapeDtypeStruct((NPAD, 1), jnp.float32)
    hspec = pl.BlockSpec((32, _BLK), lambda i: (0, i))
    cspec = pl.BlockSpec((_BLK, 1), lambda i: (i, 0))
    return pl.pallas_call(
        body,
        grid=(_NBLK,),
        in_specs=[hspec] * 3,
        out_specs=[cspec] * 3,
        out_shape=(sh, sh, sh),
    )(huu, hcu, huc)


def _newf_tc(ufp, etabs):
    nb = ufp.shape[0] // _BLK

    def body(uf_r, e0_r, e3_r, e7_r, e8_r, e9_r, out_r):
        uf = uf_r[...]
        ers = (e0_r, e3_r, e7_r, e8_r, e9_r)

        def emb(col, er):
            e = er[...]
            b = uf[:, col:col + 1]
            return jnp.where(b > 0.5, e[1:2, :25], e[0:1, :25])

        out_r[...] = jnp.concatenate(
            [emb(0, ers[0]), uf[:, 1:3], emb(3, ers[1]), uf[:, 4:7],
             emb(7, ers[2]), emb(8, ers[3]), emb(9, ers[4]),
             jnp.zeros((_BLK, 14), jnp.float32)], axis=1)

    return pl.pallas_call(
        body,
        grid=(nb,),
        in_specs=[_row_spec(16)] + [_full_spec((8, 128))] * 5,
        out_specs=_row_spec(DU0),
        out_shape=jax.ShapeDtypeStruct((ufp.shape[0], DU0), jnp.float32),
    )(ufp, *etabs)


def _scale_rows_tc(x, dinv):
    d = x.shape[1]

    def body(x_r, s_r, o_r):
        o_r[...] = x_r[...] * s_r[...]

    return pl.pallas_call(
        body,
        grid=(_NBLK,),
        in_specs=[_row_spec(d), pl.BlockSpec((_BLK, 1), lambda i: (i, 0))],
        out_specs=_row_spec(d),
        out_shape=jax.ShapeDtypeStruct((NPAD, d), jnp.float32),
    )(x, dinv)


def _leaky(x):
    return jnp.where(x >= 0, x, 0.3 * x)


def _hop0_tc(agg_uu, agg_cu, agg_uc, x_user, x_com, dinv, iccu, icuc, p):
    gcn_w = jnp.pad(p["gcn_W0"], ((0, 14), (0, 0)))
    cu_wl = p["cu_Wl0"]
    cu_wr = jnp.pad(p["cu_Wr0"], ((0, 14), (0, 0)))
    uc_wl = jnp.pad(p["uc_Wl0"], ((0, 14), (0, 0)))
    uc_wr = p["uc_Wr0"]
    gcn_b = p["gcn_b0"].reshape(1, HID)
    cu_bl = p["cu_bl0"].reshape(1, HID)
    uc_bl = p["uc_bl0"].reshape(1, HID)

    def body(auu_r, acu_r, auc_r, xu_r, xc_r, dv_r, icc_r, icu_r,
             gw_r, cwl_r, cwr_r, uwl_r, uwr_r, gb_r, cb_r, ub_r,
             ou_r, xs_r, oc_r):
        du = dv_r[...]
        u = (du * jnp.dot(auu_r[...], gw_r[...],
                          preferred_element_type=jnp.float32)
             + icc_r[...] * jnp.dot(acu_r[...], cwl_r[...],
                                    preferred_element_type=jnp.float32)
             + jnp.dot(xu_r[...], cwr_r[...],
                       preferred_element_type=jnp.float32)
             + gb_r[...] + cb_r[...])
        ou = _leaky(u)
        ou_r[...] = ou
        xs_r[...] = du * ou
        cm = (icu_r[...] * jnp.dot(auc_r[...], uwl_r[...],
                                   preferred_element_type=jnp.float32)
              + jnp.dot(xc_r[...], uwr_r[...],
                        preferred_element_type=jnp.float32)
              + ub_r[...])
        oc_r[...] = _leaky(cm)

    sh = jax.ShapeDtypeStruct((NPAD, HID), jnp.float32)
    return pl.pallas_call(
        body,
        grid=(_NBLK,),
        in_specs=[_row_spec(DU0), _row_spec(DC0), _row_spec(DU0),
                  _row_spec(DU0), _row_spec(DC0),
                  pl.BlockSpec((_BLK, 1), lambda i: (i, 0)),
                  pl.BlockSpec((_BLK, 1), lambda i: (i, 0)),
                  pl.BlockSpec((_BLK, 1), lambda i: (i, 0)),
                  _full_spec((DU0, HID)), _full_spec((DC0, HID)),
                  _full_spec((DU0, HID)), _full_spec((DU0, HID)),
                  _full_spec((DC0, HID)),
                  _full_spec((1, HID)), _full_spec((1, HID)),
                  _full_spec((1, HID))],
        out_specs=[_row_spec(HID)] * 3,
        out_shape=(sh, sh, sh),
    )(agg_uu, agg_cu, agg_uc, x_user, x_com, dinv, iccu, icuc,
      gcn_w, cu_wl, cu_wr, uc_wl, uc_wr, gcn_b, cu_bl, uc_bl)


def _hop1_tc(agg_uu, agg_cu, u0, dinv, iccu, p):
    gcn_b = p["gcn_b1"].reshape(1, HID)
    cu_bl = p["cu_bl1"].reshape(1, HID)

    def body(auu_r, acu_r, u0_r, dv_r, icc_r,
             gw_r, cwl_r, cwr_r, gb_r, cb_r,
             node_r, ps_r, ps2_r):
        du = dv_r[...]
        u = (du * jnp.dot(auu_r[...], gw_r[...],
                          preferred_element_type=jnp.float32)
             + icc_r[...] * jnp.dot(acu_r[...], cwl_r[...],
                                    preferred_element_type=jnp.float32)
             + jnp.dot(u0_r[...], cwr_r[...],
                       preferred_element_type=jnp.float32)
             + gb_r[...] + cb_r[...])
        node = u0_r[...] + _leaky(u)
        node_r[...] = node
        rid = (pl.program_id(0) * _BLK
               + lax.broadcasted_iota(jnp.int32, (_BLK, 1), 0))
        nm = jnp.where(rid < NU, node, 0.0)
        z7 = jnp.zeros((7, HID), jnp.float32)
        ps_r[...] = jnp.concatenate(
            [jnp.sum(nm, axis=0, keepdims=True), z7], axis=0)
        ps2_r[...] = jnp.concatenate(
            [jnp.sum(nm * nm, axis=0, keepdims=True), z7], axis=0)

    return pl.pallas_call(
        body,
        grid=(_NBLK,),
        in_specs=[_row_spec(HID)] * 3
        + [pl.BlockSpec((_BLK, 1), lambda i: (i, 0))] * 2
        + [_full_spec((HID, HID))] * 3 + [_full_spec((1, HID))] * 2,
        out_specs=[_row_spec(HID),
                   pl.BlockSpec((8, HID), lambda i: (i, 0)),
                   pl.BlockSpec((8, HID), lambda i: (i, 0))],
        out_shape=(jax.ShapeDtypeStruct((NPAD, HID), jnp.float32),
                   jax.ShapeDtypeStruct((8 * _NBLK, HID), jnp.float32),
                   jax.ShapeDtypeStruct((8 * _NBLK, HID), jnp.float32)),
    )(agg_uu, agg_cu, u0, dinv, iccu,
      p["gcn_W1"], p["cu_Wl1"], p["cu_Wr1"], gcn_b, cu_bl)


def _bn_coef_tc(ps, ps2, g, b):
    def body(ps_r, ps2_r, g_r, b_r, coef_r):
        m = jnp.sum(ps_r[...], axis=0, keepdims=True) / NU
        v = jnp.sum(ps2_r[...], axis=0, keepdims=True) / NU - m * m
        scale = g_r[...] / jnp.sqrt(v + 1e-5)
        shift = b_r[...] - m * scale
        coef_r[...] = jnp.concatenate(
            [scale, shift, jnp.zeros((6, HID), jnp.float32)], axis=0)

    return pl.pallas_call(
        body,
        out_shape=jax.ShapeDtypeStruct((8, HID), jnp.float32),
    )(ps, ps2, g.reshape(1, HID), b.reshape(1, HID))


def _bn_apply_tc(node, coef):
    def body(n_r, c_r, o_r):
        cf = c_r[...]
        o_r[...] = n_r[...] * cf[0:1, :] + cf[1:2, :]

    return pl.pallas_call(
        body,
        grid=(_NBLK,),
        in_specs=[_row_spec(HID), _full_spec((8, HID))],
        out_specs=_row_spec(HID),
        out_shape=jax.ShapeDtypeStruct((NPAD, HID), jnp.float32),
    )(node, coef)


def _pad_edges(ei, ep):
    src = jnp.pad(ei[0].astype(jnp.int32), (0, ep - ei.shape[1]))
    dst = jnp.pad(ei[1].astype(jnp.int32), (0, ep - ei.shape[1]),
                  constant_values=NPAD)
    return src, dst


def kernel(no_Nidx, u_feature, comment_x, edge_uu, edge_uc, edge_cu, params):
    p = params
    src_uu, dst_uu = _pad_edges(edge_uu, EP_UU)
    src_uc, dst_uc = _pad_edges(edge_uc, EP_UC)
    src_cu, dst_cu = _pad_edges(edge_cu, EP_UC)

    def compact(src, dst, ep, cap):
        lo = _make_compact(ep // 32, cap, 4704, 0)(src, dst)
        hi = _make_compact(ep // 32, cap, 4704, NPC)(src, dst)
        ss = jnp.concatenate([lo[0], hi[0]], axis=0)
        dd = jnp.concatenate([lo[1], hi[1]], axis=0)
        return ss, dd, lo[2] + hi[2]

    suu, duu, cuu = compact(src_uu, dst_uu, EP_UU, CAP_UU)
    suc, duc, cuc = compact(src_uc, dst_uc, EP_UC, CAP_UC)
    scu, dcu, ccu = compact(src_cu, dst_cu, EP_UC, CAP_UC)

    hist_uu = _make_hist(EP_UU // 32, 4704)(dst_uu)
    hist_cu = _make_hist(EP_UC // 32, 4704)(dst_cu)
    hist_uc = _make_hist(EP_UC // 32, 4704)(dst_uc)
    dinv, iccu, icuc = _scales_tc(hist_uu, hist_cu, hist_uc)

    nidx = jnp.pad(no_Nidx.astype(jnp.int32), (0, 28672 - no_Nidx.shape[0]))
    emb_p = jnp.pad(params["emb"], ((0, 0), (0, DU0 - params["emb"].shape[1])))
    prof = _make_gather(DU0, 28672, 896)(emb_p, nidx)
    ufp = jnp.pad(u_feature, ((0, 25088 - u_feature.shape[0]), (0, 6)))
    etabs = [jnp.pad(p[n], ((0, 6), (0, 103))) for n in
             ("e0", "e3", "e7", "e8", "e9")]
    newf = _newf_tc(ufp, etabs)
    x_user0 = jnp.pad(
        jnp.concatenate([newf[:25000], prof[:25000]], axis=0),
        ((0, NPAD - NU), (0, 0)))
    xs0 = _scale_rows_tc(x_user0, dinv)
    x_com0 = jnp.pad(comment_x, ((0, NPAD - NU), (0, 0)))

    seg144_uu = _make_segsum(DU0, CAP_UU)
    seg144_xc = _make_segsum(DU0, CAP_UC)
    seg64_xc = _make_segsum(DC0, CAP_UC)
    agg_uu0 = seg144_uu(xs0, suu, duu, cuu)
    agg_cu0 = seg64_xc(x_com0, scu, dcu, ccu)
    agg_uc0 = seg144_xc(x_user0, suc, duc, cuc)
    ou0, xs1, oc0 = _hop0_tc(agg_uu0, agg_cu0, agg_uc0, x_user0, x_com0,
                             dinv, iccu, icuc, p)

    seg128_uu = _make_segsum(HID, CAP_UU)
    seg128_xc = _make_segsum(HID, CAP_UC)
    agg_uu1 = seg128_uu(xs1, suu, duu, cuu)
    agg_cu1 = seg128_xc(oc0, scu, dcu, ccu)
    node, ps, ps2 = _hop1_tc(agg_uu1, agg_cu1, ou0, dinv, iccu, p)

    coef = _bn_coef_tc(ps, ps2, p["bn_g"], p["bn_b"])
    out = _bn_apply_tc(node, coef)
    return out[:NU]


# speedup vs baseline: 1.8191x; 1.8191x over previous
import functools

import jax
import jax.numpy as jnp
from jax import lax
from jax.experimental import pallas as pl
from jax.experimental.pallas import tpu as pltpu
from jax.experimental.pallas import tpu_sc as plsc

NU = 50000
NPAD = 50688
NP = 6
NPC = 3
R = 8448
RG = 8448
ACC_ROWS = 8464
STRIPE = 528
CHUNK = 128
ZR = 48

E_UU = 300000
E_UC = 150000
EP_UU = 301056
EP_UC = 150528
CAP_UU = 9728
CAP_UC = 5120
HID = 128
DU0 = 144
DC0 = 64


def _mesh():
    return plsc.VectorSubcoreMesh(core_axis_name="c", subcore_axis_name="s")


def _make_compact(eperw, cap, be, plo):
    nchunks = eperw // be

    @functools.partial(
        pl.kernel,
        out_type=(
            jax.ShapeDtypeStruct((NPC, 32, cap), jnp.int32),
            jax.ShapeDtypeStruct((NPC, 32, cap), jnp.int32),
            jax.ShapeDtypeStruct((32, 16), jnp.int32),
        ),
        mesh=_mesh(),
        compiler_params=pltpu.CompilerParams(use_tc_tiling_on_sc=False, needs_layout_passes=False),
        scratch_types=[
            pltpu.VMEM((be,), jnp.int32),
            pltpu.VMEM((be,), jnp.int32),
            pltpu.VMEM((cap,), jnp.int32),
            pltpu.VMEM((cap,), jnp.int32),
            pltpu.VMEM((cap,), jnp.int32),
            pltpu.VMEM((cap,), jnp.int32),
            pltpu.VMEM((cap,), jnp.int32),
            pltpu.VMEM((cap,), jnp.int32),
            pltpu.VMEM((16,), jnp.int32),
        ],
    )
    def k(src_h, dst_h, srcs_o, dsts_o, cnt_o,
          es_v, ed_v, sb0, sb1, sb2, db0, db1, db2, cvec):
        sb = (sb0, sb1, sb2)
        db = (db0, db1, db2)
        w = lax.axis_index("c") * 16 + lax.axis_index("s")
        base = w * eperw
        lane = lax.iota(jnp.int32, 16)
        offs = (jnp.int32(0),) * NPC
        for ci in range(nchunks):
            pltpu.sync_copy(src_h.at[pl.ds(base + ci * be, be)], es_v)
            pltpu.sync_copy(dst_h.at[pl.ds(base + ci * be, be)], ed_v)

            def body(g, offs):
                sv = es_v[pl.ds(g * 16, 16)]
                dv = ed_v[pl.ds(g * 16, 16)]
                new = []
                for i in range(NPC):
                    lo = (plo + i) * R
                    m = (dv >= lo) & (dv < lo + R)
                    inc = plsc.cumsum(m.astype(jnp.int32))
                    pos = offs[i] + inc - 1
                    plsc.store_scatter(sb[i], [pos], sv, mask=m)
                    plsc.store_scatter(db[i], [pos], dv - lo, mask=m)
                    new.append(offs[i] + jnp.sum(m.astype(jnp.int32)))
                return tuple(new)

            offs = lax.fori_loop(0, be // 16, body, offs)
        zs = jnp.zeros((16,), jnp.int32)
        gs = jnp.full((16,), RG, jnp.int32)
        for i in range(NPC):
            for j in range(256 // 16):
                sb[i][pl.ds(offs[i] + j * 16, 16)] = zs
                db[i][pl.ds(offs[i] + j * 16, 16)] = gs
        cv = jnp.zeros((16,), jnp.int32)
        for i in range(NPC):
            cv = jnp.where(lane == (plo + i), offs[i], cv)
        cvec[...] = cv
        pltpu.sync_copy(cvec, cnt_o.at[w])
        for i in range(NPC):
            pltpu.sync_copy(sb[i], srcs_o.at[i, w])
            pltpu.sync_copy(db[i], dsts_o.at[i, w])

    return k


def _make_segsum(d, cap):
    nch = cap // CHUNK

    @functools.partial(
        pl.kernel,
        out_type=jax.ShapeDtypeStruct((NPAD, d), jnp.float32),
        mesh=_mesh(),
        compiler_params=pltpu.CompilerParams(use_tc_tiling_on_sc=False, needs_layout_passes=False),
        scratch_types=[
            pltpu.VMEM_SHARED((ACC_ROWS, d), jnp.float32),
            pltpu.VMEM((nch, 128), jnp.int32),
            pltpu.VMEM((nch, 128), jnp.int32),
            pltpu.VMEM((CHUNK, d), jnp.float32),
            pltpu.VMEM((ZR, d), jnp.float32),
            pltpu.VMEM((16,), jnp.int32),
            pltpu.SemaphoreType.DMA,
        ],
    )
    def k(table, srcs, dsts, cnts, agg,
          acc_sh, src_v, dst_v, rows_v, zbuf, cvec, sem):
        c = lax.axis_index("c")
        s = lax.axis_index("s")
        lane = lax.iota(jnp.int32, 16)
        zv = jnp.zeros((16,), jnp.float32)

        def zb(r, carry):
            for j in range(d // 16):
                zbuf[r, pl.ds(j * 16, 16)] = zv
            return carry

        lax.fori_loop(0, ZR, zb, 0)
        for k2 in range(NPC):
            p = NPC * c + k2

            def zrow(t, carry):
                pltpu.sync_copy(
                    zbuf, acc_sh.at[pl.ds(s * STRIPE + t * ZR, ZR)])
                return carry

            lax.fori_loop(0, STRIPE // ZR, zrow, 0)
            plsc.subcore_barrier()
            for sk in range(2):
                w = sk * 16 + s
                pltpu.sync_copy(cnts.at[w], cvec)
                cnt = jnp.sum(jnp.where(lane == p, cvec[...], 0))
                nchunks = (cnt + (CHUNK - 1)) >> 7

                pltpu.sync_copy(srcs.at[p, w], src_v)
                pltpu.sync_copy(dsts.at[p, w], dst_v)

                def chunk(j, carry):
                    pltpu.async_copy(
                        table.at[src_v.at[j]], rows_v, sem).wait()
                    pltpu.sync_copy(
                        rows_v, acc_sh.at[dst_v.at[j]], add=True)
                    return carry

                lax.fori_loop(0, nchunks, chunk, 0)
            plsc.subcore_barrier()
            pltpu.sync_copy(
                acc_sh.at[pl.ds(s * STRIPE, STRIPE)],
                agg.at[pl.ds(p * R + s * STRIPE, STRIPE)])
            plsc.subcore_barrier()

    return k


def _make_hist(eperw, be):
    nchunks = eperw // be
    hsz = NPAD + 16

    @functools.partial(
        pl.kernel,
        out_type=jax.ShapeDtypeStruct((32, NPAD), jnp.float32),
        mesh=_mesh(),
        compiler_params=pltpu.CompilerParams(use_tc_tiling_on_sc=False, needs_layout_passes=False),
        scratch_types=[
            pltpu.VMEM((be,), jnp.int32),
            pltpu.VMEM((hsz,), jnp.float32),
        ],
    )
    def k(dst_h, out, ed_v, hist_v):
        w = lax.axis_index("c") * 16 + lax.axis_index("s")
        base = w * eperw
        zv = jnp.zeros((16,), jnp.float32)
        ov = jnp.ones((16,), jnp.float32)

        def zb(r, carry):
            hist_v[pl.ds(r * 16, 16)] = zv
            return carry

        lax.fori_loop(0, hsz // 16, zb, 0)
        for ci in range(nchunks):
            pltpu.sync_copy(dst_h.at[pl.ds(base + ci * be, be)], ed_v)

            def body(g, carry):
                dv = ed_v[pl.ds(g * 16, 16)]
                plsc.addupdate_scatter(hist_v, [dv], ov)
                return carry

            lax.fori_loop(0, be // 16, body, 0)
        pltpu.sync_copy(hist_v.at[pl.ds(0, NPAD)], out.at[w])

    return k


def _make_gather(d, b, perw):
    nch = perw // CHUNK

    @functools.partial(
        pl.kernel,
        out_type=jax.ShapeDtypeStruct((b, d), jnp.float32),
        mesh=_mesh(),
        compiler_params=pltpu.CompilerParams(use_tc_tiling_on_sc=False, needs_layout_passes=False),
        scratch_types=[
            pltpu.VMEM((perw,), jnp.int32),
            pltpu.VMEM((CHUNK, d), jnp.float32),
            pltpu.SemaphoreType.DMA,
        ],
    )
    def k(table, idx_h, out, idx_v, rows_v, sem):
        w = lax.axis_index("c") * 16 + lax.axis_index("s")
        pltpu.sync_copy(idx_h.at[pl.ds(w * perw, perw)], idx_v)
        for j in range(nch):
            pltpu.async_copy(
                table.at[idx_v.at[pl.ds(j * CHUNK, CHUNK)]],
                rows_v, sem).wait()
            pltpu.sync_copy(rows_v, out.at[pl.ds(w * perw + j * CHUNK, CHUNK)])

    return k


_BLK = 128
_NBLK = NPAD // _BLK


def _row_spec(d):
    return pl.BlockSpec((_BLK, d), lambda i: (i, 0))


def _full_spec(shape):
    return pl.BlockSpec(shape, lambda i: tuple(0 for _ in shape))


def _col(v):
    i = lax.broadcasted_iota(jnp.int32, (_BLK, _BLK), 0)
    j = lax.broadcasted_iota(jnp.int32, (_BLK, _BLK), 1)
    diag = jnp.where(i == j, jnp.broadcast_to(v, (_BLK, _BLK)), 0.0)
    return jnp.sum(diag, axis=1, keepdims=True)


def _scales_tc(huu, hcu, huc):
    def body(huu_r, hcu_r, huc_r, dinv_r, iccu_r, icuc_r):
        dg = _col(jnp.sum(huu_r[...], axis=0, keepdims=True))
        cc = _col(jnp.sum(hcu_r[...], axis=0, keepdims=True))
        cu = _col(jnp.sum(huc_r[...], axis=0, keepdims=True))
        dinv_r[...] = jnp.where(dg > 0, lax.rsqrt(jnp.maximum(dg, 1.0)), 0.0)
        iccu_r[...] = 1.0 / jnp.maximum(cc, 1.0)
        icuc_r[...] = 1.0 / jnp.maximum(cu, 1.0)

    sh = jax.ShapeDtypeStruct((NPAD, 1), jnp.float32)
    hspec = pl.BlockSpec((32, _BLK), lambda i: (0, i))
    cspec = pl.BlockSpec((_BLK, 1), lambda i: (i, 0))
    return pl.pallas_call(
        body,
        grid=(_NBLK,),
        in_specs=[hspec] * 3,
        out_specs=[cspec] * 3,
        out_shape=(sh, sh, sh),
    )(huu, hcu, huc)


def _newf_tc(ufp, etabs):
    nb = ufp.shape[0] // _BLK

    def body(uf_r, e0_r, e3_r, e7_r, e8_r, e9_r, out_r):
        uf = uf_r[...]
        ers = (e0_r, e3_r, e7_r, e8_r, e9_r)

        def emb(col, er):
            e = er[...]
            b = uf[:, col:col + 1]
            return jnp.where(b > 0.5, e[1:2, :25], e[0:1, :25])

        out_r[...] = jnp.concatenate(
            [emb(0, ers[0]), uf[:, 1:3], emb(3, ers[1]), uf[:, 4:7],
             emb(7, ers[2]), emb(8, ers[3]), emb(9, ers[4]),
             jnp.zeros((_BLK, 14), jnp.float32)], axis=1)

    return pl.pallas_call(
        body,
        grid=(nb,),
        in_specs=[_row_spec(16)] + [_full_spec((8, 128))] * 5,
        out_specs=_row_spec(DU0),
        out_shape=jax.ShapeDtypeStruct((ufp.shape[0], DU0), jnp.float32),
    )(ufp, *etabs)


def _scale_rows_tc(x, dinv):
    d = x.shape[1]

    def body(x_r, s_r, o_r):
        o_r[...] = x_r[...] * s_r[...]

    return pl.pallas_call(
        body,
        grid=(_NBLK,),
        in_specs=[_row_spec(d), pl.BlockSpec((_BLK, 1), lambda i: (i, 0))],
        out_specs=_row_spec(d),
        out_shape=jax.ShapeDtypeStruct((NPAD, d), jnp.float32),
    )(x, dinv)


def _leaky(x):
    return jnp.where(x >= 0, x, 0.3 * x)


def _hop0_tc(agg_uu, agg_cu, agg_uc, x_user, x_com, dinv, iccu, icuc, p):
    gcn_w = jnp.pad(p["gcn_W0"], ((0, 14), (0, 0)))
    cu_wl = p["cu_Wl0"]
    cu_wr = jnp.pad(p["cu_Wr0"], ((0, 14), (0, 0)))
    uc_wl = jnp.pad(p["uc_Wl0"], ((0, 14), (0, 0)))
    uc_wr = p["uc_Wr0"]
    gcn_b = p["gcn_b0"].reshape(1, HID)
    cu_bl = p["cu_bl0"].reshape(1, HID)
    uc_bl = p["uc_bl0"].reshape(1, HID)

    def body(auu_r, acu_r, auc_r, xu_r, xc_r, dv_r, icc_r, icu_r,
             gw_r, cwl_r, cwr_r, uwl_r, uwr_r, gb_r, cb_r, ub_r,
             ou_r, xs_r, oc_r):
        du = dv_r[...]
        u = (du * jnp.dot(auu_r[...], gw_r[...],
                          preferred_element_type=jnp.float32)
             + icc_r[...] * jnp.dot(acu_r[...], cwl_r[...],
                                    preferred_element_type=jnp.float32)
             + jnp.dot(xu_r[...], cwr_r[...],
                       preferred_element_type=jnp.float32)
             + gb_r[...] + cb_r[...])
        ou = _leaky(u)
        ou_r[...] = ou
        xs_r[...] = du * ou
        cm = (icu_r[...] * jnp.dot(auc_r[...], uwl_r[...],
                                   preferred_element_type=jnp.float32)
              + jnp.dot(xc_r[...], uwr_r[...],
                        preferred_element_type=jnp.float32)
              + ub_r[...])
        oc_r[...] = _leaky(cm)

    sh = jax.ShapeDtypeStruct((NPAD, HID), jnp.float32)
    return pl.pallas_call(
        body,
        grid=(_NBLK,),
        in_specs=[_row_spec(DU0), _row_spec(DC0), _row_spec(DU0),
                  _row_spec(DU0), _row_spec(DC0),
                  pl.BlockSpec((_BLK, 1), lambda i: (i, 0)),
                  pl.BlockSpec((_BLK, 1), lambda i: (i, 0)),
                  pl.BlockSpec((_BLK, 1), lambda i: (i, 0)),
                  _full_spec((DU0, HID)), _full_spec((DC0, HID)),
                  _full_spec((DU0, HID)), _full_spec((DU0, HID)),
                  _full_spec((DC0, HID)),
                  _full_spec((1, HID)), _full_spec((1, HID)),
                  _full_spec((1, HID))],
        out_specs=[_row_spec(HID)] * 3,
        out_shape=(sh, sh, sh),
    )(agg_uu, agg_cu, agg_uc, x_user, x_com, dinv, iccu, icuc,
      gcn_w, cu_wl, cu_wr, uc_wl, uc_wr, gcn_b, cu_bl, uc_bl)


def _hop1_tc(agg_uu, agg_cu, u0, dinv, iccu, p):
    gcn_b = p["gcn_b1"].reshape(1, HID)
    cu_bl = p["cu_bl1"].reshape(1, HID)

    def body(auu_r, acu_r, u0_r, dv_r, icc_r,
             gw_r, cwl_r, cwr_r, gb_r, cb_r,
             node_r, ps_r, ps2_r):
        du = dv_r[...]
        u = (du * jnp.dot(auu_r[...], gw_r[...],
                          preferred_element_type=jnp.float32)
             + icc_r[...] * jnp.dot(acu_r[...], cwl_r[...],
                                    preferred_element_type=jnp.float32)
             + jnp.dot(u0_r[...], cwr_r[...],
                       preferred_element_type=jnp.float32)
             + gb_r[...] + cb_r[...])
        node = u0_r[...] + _leaky(u)
        node_r[...] = node
        rid = (pl.program_id(0) * _BLK
               + lax.broadcasted_iota(jnp.int32, (_BLK, 1), 0))
        nm = jnp.where(rid < NU, node, 0.0)
        z7 = jnp.zeros((7, HID), jnp.float32)
        ps_r[...] = jnp.concatenate(
            [jnp.sum(nm, axis=0, keepdims=True), z7], axis=0)
        ps2_r[...] = jnp.concatenate(
            [jnp.sum(nm * nm, axis=0, keepdims=True), z7], axis=0)

    return pl.pallas_call(
        body,
        grid=(_NBLK,),
        in_specs=[_row_spec(HID)] * 3
        + [pl.BlockSpec((_BLK, 1), lambda i: (i, 0))] * 2
        + [_full_spec((HID, HID))] * 3 + [_full_spec((1, HID))] * 2,
        out_specs=[_row_spec(HID),
                   pl.BlockSpec((8, HID), lambda i: (i, 0)),
                   pl.BlockSpec((8, HID), lambda i: (i, 0))],
        out_shape=(jax.ShapeDtypeStruct((NPAD, HID), jnp.float32),
                   jax.ShapeDtypeStruct((8 * _NBLK, HID), jnp.float32),
                   jax.ShapeDtypeStruct((8 * _NBLK, HID), jnp.float32)),
    )(agg_uu, agg_cu, u0, dinv, iccu,
      p["gcn_W1"], p["cu_Wl1"], p["cu_Wr1"], gcn_b, cu_bl)


def _bn_coef_tc(ps, ps2, g, b):
    def body(ps_r, ps2_r, g_r, b_r, coef_r):
        m = jnp.sum(ps_r[...], axis=0, keepdims=True) / NU
        v = jnp.sum(ps2_r[...], axis=0, keepdims=True) / NU - m * m
        scale = g_r[...] / jnp.sqrt(v + 1e-5)
        shift = b_r[...] - m * scale
        coef_r[...] = jnp.concatenate(
            [scale, shift, jnp.zeros((6, HID), jnp.float32)], axis=0)

    return pl.pallas_call(
        body,
        out_shape=jax.ShapeDtypeStruct((8, HID), jnp.float32),
    )(ps, ps2, g.reshape(1, HID), b.reshape(1, HID))


def _bn_apply_tc(node, coef):
    def body(n_r, c_r, o_r):
        cf = c_r[...]
        o_r[...] = n_r[...] * cf[0:1, :] + cf[1:2, :]

    return pl.pallas_call(
        body,
        grid=(_NBLK,),
        in_specs=[_row_spec(HID), _full_spec((8, HID))],
        out_specs=_row_spec(HID),
        out_shape=jax.ShapeDtypeStruct((NPAD, HID), jnp.float32),
    )(node, coef)


def _pad_edges(ei, ep):
    src = jnp.pad(ei[0].astype(jnp.int32), (0, ep - ei.shape[1]))
    dst = jnp.pad(ei[1].astype(jnp.int32), (0, ep - ei.shape[1]),
                  constant_values=NPAD)
    return src, dst


def kernel(no_Nidx, u_feature, comment_x, edge_uu, edge_uc, edge_cu, params):
    p = params
    src_uu, dst_uu = _pad_edges(edge_uu, EP_UU)
    src_uc, dst_uc = _pad_edges(edge_uc, EP_UC)
    src_cu, dst_cu = _pad_edges(edge_cu, EP_UC)

    def compact(src, dst, ep, cap):
        lo = _make_compact(ep // 32, cap, 4704, 0)(src, dst)
        hi = _make_compact(ep // 32, cap, 4704, NPC)(src, dst)
        n = cap // CHUNK
        ss = jnp.concatenate([lo[0], hi[0]], axis=0).reshape(NP, 32, n, 128)
        dd = jnp.concatenate([lo[1], hi[1]], axis=0).reshape(NP, 32, n, 128)
        return ss, dd, lo[2] + hi[2]

    suu, duu, cuu = compact(src_uu, dst_uu, EP_UU, CAP_UU)
    suc, duc, cuc = compact(src_uc, dst_uc, EP_UC, CAP_UC)
    scu, dcu, ccu = compact(src_cu, dst_cu, EP_UC, CAP_UC)

    hist_uu = _make_hist(EP_UU // 32, 4704)(dst_uu)
    hist_cu = _make_hist(EP_UC // 32, 4704)(dst_cu)
    hist_uc = _make_hist(EP_UC // 32, 4704)(dst_uc)
    dinv, iccu, icuc = _scales_tc(hist_uu, hist_cu, hist_uc)

    nidx = jnp.pad(no_Nidx.astype(jnp.int32), (0, 28672 - no_Nidx.shape[0]))
    emb_p = jnp.pad(params["emb"], ((0, 0), (0, DU0 - params["emb"].shape[1])))
    prof = _make_gather(DU0, 28672, 896)(emb_p, nidx)
    ufp = jnp.pad(u_feature, ((0, 25088 - u_feature.shape[0]), (0, 6)))
    etabs = [jnp.pad(p[n], ((0, 6), (0, 103))) for n in
             ("e0", "e3", "e7", "e8", "e9")]
    newf = _newf_tc(ufp, etabs)
    x_user0 = jnp.pad(
        jnp.concatenate([newf[:25000], prof[:25000]], axis=0),
        ((0, NPAD - NU), (0, 0)))
    xs0 = _scale_rows_tc(x_user0, dinv)
    x_com0 = jnp.pad(comment_x, ((0, NPAD - NU), (0, 0)))

    seg144_uu = _make_segsum(DU0, CAP_UU)
    seg144_xc = _make_segsum(DU0, CAP_UC)
    seg64_xc = _make_segsum(DC0, CAP_UC)
    agg_uu0 = seg144_uu(xs0, suu, duu, cuu)
    agg_cu0 = seg64_xc(x_com0, scu, dcu, ccu)
    agg_uc0 = seg144_xc(x_user0, suc, duc, cuc)
    ou0, xs1, oc0 = _hop0_tc(agg_uu0, agg_cu0, agg_uc0, x_user0, x_com0,
                             dinv, iccu, icuc, p)

    seg128_uu = _make_segsum(HID, CAP_UU)
    seg128_xc = _make_segsum(HID, CAP_UC)
    agg_uu1 = seg128_uu(xs1, suu, duu, cuu)
    agg_cu1 = seg128_xc(oc0, scu, dcu, ccu)
    node, ps, ps2 = _hop1_tc(agg_uu1, agg_cu1, ou0, dinv, iccu, p)

    coef = _bn_coef_tc(ps, ps2, p["bn_g"], p["bn_b"])
    out = _bn_apply_tc(node, coef)
    return out[:NU]


# speedup vs baseline: 1.8266x; 1.0041x over previous
import functools

import jax
import jax.numpy as jnp
from jax import lax
from jax.experimental import pallas as pl
from jax.experimental.pallas import tpu as pltpu
from jax.experimental.pallas import tpu_sc as plsc

NU = 50000
NPAD = 50688
NP = 6
NPC = 3
R = 8448
RG = 8448
ACC_ROWS = 8464
STRIPE = 528
CHUNK = 128
ZR = 48

E_UU = 300000
E_UC = 150000
EP_UU = 301056
EP_UC = 150528
CAP_UU = 9728
CAP_UC = 5120
HID = 128
DU0 = 144
DC0 = 64


def _mesh():
    return plsc.VectorSubcoreMesh(core_axis_name="c", subcore_axis_name="s")


def _make_compact(eperw, cap, be, plo):
    nchunks = eperw // be

    @functools.partial(
        pl.kernel,
        out_type=(
            jax.ShapeDtypeStruct((NPC, 32, cap), jnp.int32),
            jax.ShapeDtypeStruct((NPC, 32, cap), jnp.int32),
            jax.ShapeDtypeStruct((32, 16), jnp.int32),
        ),
        mesh=_mesh(),
        compiler_params=pltpu.CompilerParams(use_tc_tiling_on_sc=False, needs_layout_passes=False),
        scratch_types=[
            pltpu.VMEM((be,), jnp.int32),
            pltpu.VMEM((be,), jnp.int32),
            pltpu.VMEM((cap,), jnp.int32),
            pltpu.VMEM((cap,), jnp.int32),
            pltpu.VMEM((cap,), jnp.int32),
            pltpu.VMEM((cap,), jnp.int32),
            pltpu.VMEM((cap,), jnp.int32),
            pltpu.VMEM((cap,), jnp.int32),
            pltpu.VMEM((16,), jnp.int32),
        ],
    )
    def k(src_h, dst_h, srcs_o, dsts_o, cnt_o,
          es_v, ed_v, sb0, sb1, sb2, db0, db1, db2, cvec):
        sb = (sb0, sb1, sb2)
        db = (db0, db1, db2)
        w = lax.axis_index("c") * 16 + lax.axis_index("s")
        base = w * eperw
        lane = lax.iota(jnp.int32, 16)
        offs = (jnp.int32(0),) * NPC
        for ci in range(nchunks):
            pltpu.sync_copy(src_h.at[pl.ds(base + ci * be, be)], es_v)
            pltpu.sync_copy(dst_h.at[pl.ds(base + ci * be, be)], ed_v)

            def body(g, offs):
                sv = es_v[pl.ds(g * 16, 16)]
                dv = ed_v[pl.ds(g * 16, 16)]
                new = []
                for i in range(NPC):
                    lo = (plo + i) * R
                    m = (dv >= lo) & (dv < lo + R)
                    inc = plsc.cumsum(m.astype(jnp.int32))
                    pos = offs[i] + inc - 1
                    plsc.store_scatter(sb[i], [pos], sv, mask=m)
                    plsc.store_scatter(db[i], [pos], dv - lo, mask=m)
                    new.append(offs[i] + jnp.sum(m.astype(jnp.int32)))
                return tuple(new)

            offs = lax.fori_loop(0, be // 16, body, offs)
        zs = jnp.zeros((16,), jnp.int32)
        gs = jnp.full((16,), RG, jnp.int32)
        for i in range(NPC):
            for j in range(256 // 16):
                sb[i][pl.ds(offs[i] + j * 16, 16)] = zs
                db[i][pl.ds(offs[i] + j * 16, 16)] = gs
        cv = jnp.zeros((16,), jnp.int32)
        for i in range(NPC):
            cv = jnp.where(lane == (plo + i), offs[i], cv)
        cvec[...] = cv
        pltpu.sync_copy(cvec, cnt_o.at[w])
        for i in range(NPC):
            pltpu.sync_copy(sb[i], srcs_o.at[i, w])
            pltpu.sync_copy(db[i], dsts_o.at[i, w])

    return k


def _make_segsum(d, cap, nbuf=1):
    nch = cap // CHUNK

    @functools.partial(
        pl.kernel,
        out_type=jax.ShapeDtypeStruct((NPAD, d), jnp.float32),
        mesh=_mesh(),
        compiler_params=pltpu.CompilerParams(use_tc_tiling_on_sc=False, needs_layout_passes=False),
        scratch_types=[
            pltpu.VMEM_SHARED((ACC_ROWS, d), jnp.float32),
            pltpu.VMEM((nch, 128), jnp.int32),
            pltpu.VMEM((nch, 128), jnp.int32),
            pltpu.VMEM((nbuf * CHUNK, d), jnp.float32),
            pltpu.VMEM((ZR, d), jnp.float32),
            pltpu.VMEM((16,), jnp.int32),
            pltpu.SemaphoreType.DMA,
            pltpu.SemaphoreType.DMA,
        ],
    )
    def k(table, srcs, dsts, cnts, agg,
          acc_sh, src_v, dst_v, rows_v, zbuf, cvec, gsem, ssem):
        c = lax.axis_index("c")
        s = lax.axis_index("s")
        lane = lax.iota(jnp.int32, 16)
        zv = jnp.zeros((16,), jnp.float32)

        def zb(r, carry):
            for j in range(d // 16):
                zbuf[r, pl.ds(j * 16, 16)] = zv
            return carry

        lax.fori_loop(0, ZR, zb, 0)
        for k2 in range(NPC):
            p = NPC * c + k2

            def zrow(t, carry):
                pltpu.sync_copy(
                    zbuf, acc_sh.at[pl.ds(s * STRIPE + t * ZR, ZR)])
                return carry

            lax.fori_loop(0, STRIPE // ZR, zrow, 0)
            plsc.subcore_barrier()
            for sk in range(2):
                w = sk * 16 + s
                pltpu.sync_copy(cnts.at[w], cvec)
                cnt = jnp.sum(jnp.where(lane == p, cvec[...], 0))
                nchunks = (cnt + (CHUNK - 1)) >> 7

                pltpu.sync_copy(srcs.at[p, w], src_v)
                pltpu.sync_copy(dsts.at[p, w], dst_v)

                if nbuf == 1:
                    def chunk(j, carry):
                        pltpu.async_copy(
                            table.at[src_v.at[j]], rows_v, gsem).wait()
                        pltpu.sync_copy(
                            rows_v, acc_sh.at[dst_v.at[j]], add=True)
                        return carry

                    lax.fori_loop(0, nchunks, chunk, 0)
                else:
                    r0 = rows_v.at[pl.ds(0, CHUNK)]
                    r1 = rows_v.at[pl.ds(CHUNK, CHUNK)]

                    @pl.when(nchunks > 0)
                    def _():
                        pltpu.async_copy(table.at[src_v.at[0]], r0, gsem)

                    @pl.when(nchunks > 1)
                    def _():
                        pltpu.async_copy(table.at[src_v.at[1]], r1, gsem)

                    def pair(ss, carry):
                        j0 = 2 * ss
                        j1 = j0 + 1
                        pltpu.make_async_copy(
                            table.at[src_v.at[j0]], r0, gsem).wait()

                        @pl.when(j1 < nchunks)
                        def _():
                            pltpu.make_async_copy(
                                table.at[src_v.at[j1]], r1, gsem).wait()

                        pltpu.async_copy(
                            r0, acc_sh.at[dst_v.at[j0]], ssem, add=True)

                        @pl.when(j1 < nchunks)
                        def _():
                            pltpu.async_copy(
                                r1, acc_sh.at[dst_v.at[j1]], ssem, add=True)

                        pltpu.make_async_copy(
                            r0, acc_sh.at[dst_v.at[j0]], ssem).wait()

                        @pl.when(j1 < nchunks)
                        def _():
                            pltpu.make_async_copy(
                                r1, acc_sh.at[dst_v.at[j1]], ssem).wait()

                        @pl.when(j0 + 2 < nchunks)
                        def _():
                            pltpu.async_copy(
                                table.at[src_v.at[j0 + 2]], r0, gsem)

                        @pl.when(j1 + 2 < nchunks)
                        def _():
                            pltpu.async_copy(
                                table.at[src_v.at[j1 + 2]], r1, gsem)

                        return carry

                    lax.fori_loop(0, (nchunks + 1) >> 1, pair, 0)
            plsc.subcore_barrier()
            pltpu.sync_copy(
                acc_sh.at[pl.ds(s * STRIPE, STRIPE)],
                agg.at[pl.ds(p * R + s * STRIPE, STRIPE)])
            plsc.subcore_barrier()

    return k


def _make_hist(eperw, be):
    nchunks = eperw // be
    hsz = NPAD + 16

    @functools.partial(
        pl.kernel,
        out_type=jax.ShapeDtypeStruct((32, NPAD), jnp.float32),
        mesh=_mesh(),
        compiler_params=pltpu.CompilerParams(use_tc_tiling_on_sc=False, needs_layout_passes=False),
        scratch_types=[
            pltpu.VMEM((be,), jnp.int32),
            pltpu.VMEM((hsz,), jnp.float32),
        ],
    )
    def k(dst_h, out, ed_v, hist_v):
        w = lax.axis_index("c") * 16 + lax.axis_index("s")
        base = w * eperw
        zv = jnp.zeros((16,), jnp.float32)
        ov = jnp.ones((16,), jnp.float32)

        def zb(r, carry):
            hist_v[pl.ds(r * 16, 16)] = zv
            return carry

        lax.fori_loop(0, hsz // 16, zb, 0)
        for ci in range(nchunks):
            pltpu.sync_copy(dst_h.at[pl.ds(base + ci * be, be)], ed_v)

            def body(g, carry):
                dv = ed_v[pl.ds(g * 16, 16)]
                plsc.addupdate_scatter(hist_v, [dv], ov)
                return carry

            lax.fori_loop(0, be // 16, body, 0)
        pltpu.sync_copy(hist_v.at[pl.ds(0, NPAD)], out.at[w])

    return k


def _make_gather(d, b, perw):
    nch = perw // CHUNK

    @functools.partial(
        pl.kernel,
        out_type=jax.ShapeDtypeStruct((b, d), jnp.float32),
        mesh=_mesh(),
        compiler_params=pltpu.CompilerParams(use_tc_tiling_on_sc=False, needs_layout_passes=False),
        scratch_types=[
            pltpu.VMEM((perw,), jnp.int32),
            pltpu.VMEM((CHUNK, d), jnp.float32),
            pltpu.SemaphoreType.DMA,
        ],
    )
    def k(table, idx_h, out, idx_v, rows_v, sem):
        w = lax.axis_index("c") * 16 + lax.axis_index("s")
        pltpu.sync_copy(idx_h.at[pl.ds(w * perw, perw)], idx_v)
        for j in range(nch):
            pltpu.async_copy(
                table.at[idx_v.at[pl.ds(j * CHUNK, CHUNK)]],
                rows_v, sem).wait()
            pltpu.sync_copy(rows_v, out.at[pl.ds(w * perw + j * CHUNK, CHUNK)])

    return k


_BLK = 128
_NBLK = NPAD // _BLK


def _row_spec(d):
    return pl.BlockSpec((_BLK, d), lambda i: (i, 0))


def _full_spec(shape):
    return pl.BlockSpec(shape, lambda i: tuple(0 for _ in shape))


def _col(v):
    i = lax.broadcasted_iota(jnp.int32, (_BLK, _BLK), 0)
    j = lax.broadcasted_iota(jnp.int32, (_BLK, _BLK), 1)
    diag = jnp.where(i == j, jnp.broadcast_to(v, (_BLK, _BLK)), 0.0)
    return jnp.sum(diag, axis=1, keepdims=True)


def _scales_tc(huu, hcu, huc):
    def body(huu_r, hcu_r, huc_r, dinv_r, iccu_r, icuc_r):
        dg = _col(jnp.sum(huu_r[...], axis=0, keepdims=True))
        cc = _col(jnp.sum(hcu_r[...], axis=0, keepdims=True))
        cu = _col(jnp.sum(huc_r[...], axis=0, keepdims=True))
        dinv_r[...] = jnp.where(dg > 0, lax.rsqrt(jnp.maximum(dg, 1.0)), 0.0)
        iccu_r[...] = 1.0 / jnp.maximum(cc, 1.0)
        icuc_r[...] = 1.0 / jnp.maximum(cu, 1.0)

    sh = jax.ShapeDtypeStruct((NPAD, 1), jnp.float32)
    hspec = pl.BlockSpec((32, _BLK), lambda i: (0, i))
    cspec = pl.BlockSpec((_BLK, 1), lambda i: (i, 0))
    return pl.pallas_call(
        body,
        grid=(_NBLK,),
        in_specs=[hspec] * 3,
        out_specs=[cspec] * 3,
        out_shape=(sh, sh, sh),
    )(huu, hcu, huc)


def _newf_tc(ufp, etabs):
    nb = ufp.shape[0] // _BLK

    def body(uf_r, e0_r, e3_r, e7_r, e8_r, e9_r, out_r):
        uf = uf_r[...]
        ers = (e0_r, e3_r, e7_r, e8_r, e9_r)

        def emb(col, er):
            e = er[...]
            b = uf[:, col:col + 1]
            return jnp.where(b > 0.5, e[1:2, :25], e[0:1, :25])

        out_r[...] = jnp.concatenate(
            [emb(0, ers[0]), uf[:, 1:3], emb(3, ers[1]), uf[:, 4:7],
             emb(7, ers[2]), emb(8, ers[3]), emb(9, ers[4]),
             jnp.zeros((_BLK, 14), jnp.float32)], axis=1)

    return pl.pallas_call(
        body,
        grid=(nb,),
        in_specs=[_row_spec(16)] + [_full_spec((8, 128))] * 5,
        out_specs=_row_spec(DU0),
        out_shape=jax.ShapeDtypeStruct((ufp.shape[0], DU0), jnp.float32),
    )(ufp, *etabs)


def _scale_rows_tc(x, dinv):
    d = x.shape[1]

    def body(x_r, s_r, o_r):
        o_r[...] = x_r[...] * s_r[...]

    return pl.pallas_call(
        body,
        grid=(_NBLK,),
        in_specs=[_row_spec(d), pl.BlockSpec((_BLK, 1), lambda i: (i, 0))],
        out_specs=_row_spec(d),
        out_shape=jax.ShapeDtypeStruct((NPAD, d), jnp.float32),
    )(x, dinv)


def _leaky(x):
    return jnp.where(x >= 0, x, 0.3 * x)


def _hop0_tc(agg_uu, agg_cu, agg_uc, x_user, x_com, dinv, iccu, icuc, p):
    gcn_w = jnp.pad(p["gcn_W0"], ((0, 14), (0, 0)))
    cu_wl = p["cu_Wl0"]
    cu_wr = jnp.pad(p["cu_Wr0"], ((0, 14), (0, 0)))
    uc_wl = jnp.pad(p["uc_Wl0"], ((0, 14), (0, 0)))
    uc_wr = p["uc_Wr0"]
    gcn_b = p["gcn_b0"].reshape(1, HID)
    cu_bl = p["cu_bl0"].reshape(1, HID)
    uc_bl = p["uc_bl0"].reshape(1, HID)

    def body(auu_r, acu_r, auc_r, xu_r, xc_r, dv_r, icc_r, icu_r,
             gw_r, cwl_r, cwr_r, uwl_r, uwr_r, gb_r, cb_r, ub_r,
             ou_r, xs_r, oc_r):
        du = dv_r[...]
        u = (du * jnp.dot(auu_r[...], gw_r[...],
                          preferred_element_type=jnp.float32)
             + icc_r[...] * jnp.dot(acu_r[...], cwl_r[...],
                                    preferred_element_type=jnp.float32)
             + jnp.dot(xu_r[...], cwr_r[...],
                       preferred_element_type=jnp.float32)
             + gb_r[...] + cb_r[...])
        ou = _leaky(u)
        ou_r[...] = ou
        xs_r[...] = du * ou
        cm = (icu_r[...] * jnp.dot(auc_r[...], uwl_r[...],
                                   preferred_element_type=jnp.float32)
              + jnp.dot(xc_r[...], uwr_r[...],
                        preferred_element_type=jnp.float32)
              + ub_r[...])
        oc_r[...] = _leaky(cm)

    sh = jax.ShapeDtypeStruct((NPAD, HID), jnp.float32)
    return pl.pallas_call(
        body,
        grid=(_NBLK,),
        in_specs=[_row_spec(DU0), _row_spec(DC0), _row_spec(DU0),
                  _row_spec(DU0), _row_spec(DC0),
                  pl.BlockSpec((_BLK, 1), lambda i: (i, 0)),
                  pl.BlockSpec((_BLK, 1), lambda i: (i, 0)),
                  pl.BlockSpec((_BLK, 1), lambda i: (i, 0)),
                  _full_spec((DU0, HID)), _full_spec((DC0, HID)),
                  _full_spec((DU0, HID)), _full_spec((DU0, HID)),
                  _full_spec((DC0, HID)),
                  _full_spec((1, HID)), _full_spec((1, HID)),
                  _full_spec((1, HID))],
        out_specs=[_row_spec(HID)] * 3,
        out_shape=(sh, sh, sh),
    )(agg_uu, agg_cu, agg_uc, x_user, x_com, dinv, iccu, icuc,
      gcn_w, cu_wl, cu_wr, uc_wl, uc_wr, gcn_b, cu_bl, uc_bl)


def _hop1_tc(agg_uu, agg_cu, u0, dinv, iccu, p):
    gcn_b = p["gcn_b1"].reshape(1, HID)
    cu_bl = p["cu_bl1"].reshape(1, HID)

    def body(auu_r, acu_r, u0_r, dv_r, icc_r,
             gw_r, cwl_r, cwr_r, gb_r, cb_r,
             node_r, ps_r, ps2_r):
        du = dv_r[...]
        u = (du * jnp.dot(auu_r[...], gw_r[...],
                          preferred_element_type=jnp.float32)
             + icc_r[...] * jnp.dot(acu_r[...], cwl_r[...],
                                    preferred_element_type=jnp.float32)
             + jnp.dot(u0_r[...], cwr_r[...],
                       preferred_element_type=jnp.float32)
             + gb_r[...] + cb_r[...])
        node = u0_r[...] + _leaky(u)
        node_r[...] = node
        rid = (pl.program_id(0) * _BLK
               + lax.broadcasted_iota(jnp.int32, (_BLK, 1), 0))
        nm = jnp.where(rid < NU, node, 0.0)
        z7 = jnp.zeros((7, HID), jnp.float32)
        ps_r[...] = jnp.concatenate(
            [jnp.sum(nm, axis=0, keepdims=True), z7], axis=0)
        ps2_r[...] = jnp.concatenate(
            [jnp.sum(nm * nm, axis=0, keepdims=True), z7], axis=0)

    return pl.pallas_call(
        body,
        grid=(_NBLK,),
        in_specs=[_row_spec(HID)] * 3
        + [pl.BlockSpec((_BLK, 1), lambda i: (i, 0))] * 2
        + [_full_spec((HID, HID))] * 3 + [_full_spec((1, HID))] * 2,
        out_specs=[_row_spec(HID),
                   pl.BlockSpec((8, HID), lambda i: (i, 0)),
                   pl.BlockSpec((8, HID), lambda i: (i, 0))],
        out_shape=(jax.ShapeDtypeStruct((NPAD, HID), jnp.float32),
                   jax.ShapeDtypeStruct((8 * _NBLK, HID), jnp.float32),
                   jax.ShapeDtypeStruct((8 * _NBLK, HID), jnp.float32)),
    )(agg_uu, agg_cu, u0, dinv, iccu,
      p["gcn_W1"], p["cu_Wl1"], p["cu_Wr1"], gcn_b, cu_bl)


def _bn_coef_tc(ps, ps2, g, b):
    def body(ps_r, ps2_r, g_r, b_r, coef_r):
        m = jnp.sum(ps_r[...], axis=0, keepdims=True) / NU
        v = jnp.sum(ps2_r[...], axis=0, keepdims=True) / NU - m * m
        scale = g_r[...] / jnp.sqrt(v + 1e-5)
        shift = b_r[...] - m * scale
        coef_r[...] = jnp.concatenate(
            [scale, shift, jnp.zeros((6, HID), jnp.float32)], axis=0)

    return pl.pallas_call(
        body,
        out_shape=jax.ShapeDtypeStruct((8, HID), jnp.float32),
    )(ps, ps2, g.reshape(1, HID), b.reshape(1, HID))


def _bn_apply_tc(node, coef):
    def body(n_r, c_r, o_r):
        cf = c_r[...]
        o_r[...] = n_r[...] * cf[0:1, :] + cf[1:2, :]

    return pl.pallas_call(
        body,
        grid=(_NBLK,),
        in_specs=[_row_spec(HID), _full_spec((8, HID))],
        out_specs=_row_spec(HID),
        out_shape=jax.ShapeDtypeStruct((NPAD, HID), jnp.float32),
    )(node, coef)


def _pad_edges(ei, ep):
    src = jnp.pad(ei[0].astype(jnp.int32), (0, ep - ei.shape[1]))
    dst = jnp.pad(ei[1].astype(jnp.int32), (0, ep - ei.shape[1]),
                  constant_values=NPAD)
    return src, dst


def kernel(no_Nidx, u_feature, comment_x, edge_uu, edge_uc, edge_cu, params):
    p = params
    src_uu, dst_uu = _pad_edges(edge_uu, EP_UU)
    src_uc, dst_uc = _pad_edges(edge_uc, EP_UC)
    src_cu, dst_cu = _pad_edges(edge_cu, EP_UC)

    def compact(src, dst, ep, cap):
        lo = _make_compact(ep // 32, cap, 4704, 0)(src, dst)
        hi = _make_compact(ep // 32, cap, 4704, NPC)(src, dst)
        n = cap // CHUNK
        ss = jnp.concatenate([lo[0], hi[0]], axis=0).reshape(NP, 32, n, 128)
        dd = jnp.concatenate([lo[1], hi[1]], axis=0).reshape(NP, 32, n, 128)
        return ss, dd, lo[2] + hi[2]

    suu, duu, cuu = compact(src_uu, dst_uu, EP_UU, CAP_UU)
    suc, duc, cuc = compact(src_uc, dst_uc, EP_UC, CAP_UC)
    scu, dcu, ccu = compact(src_cu, dst_cu, EP_UC, CAP_UC)

    hist_uu = _make_hist(EP_UU // 32, 4704)(dst_uu)
    hist_cu = _make_hist(EP_UC // 32, 4704)(dst_cu)
    hist_uc = _make_hist(EP_UC // 32, 4704)(dst_uc)
    dinv, iccu, icuc = _scales_tc(hist_uu, hist_cu, hist_uc)

    nidx = jnp.pad(no_Nidx.astype(jnp.int32), (0, 28672 - no_Nidx.shape[0]))
    emb_p = jnp.pad(params["emb"], ((0, 0), (0, DU0 - params["emb"].shape[1])))
    prof = _make_gather(DU0, 28672, 896)(emb_p, nidx)
    ufp = jnp.pad(u_feature, ((0, 25088 - u_feature.shape[0]), (0, 6)))
    etabs = [jnp.pad(p[n], ((0, 6), (0, 103))) for n in
             ("e0", "e3", "e7", "e8", "e9")]
    newf = _newf_tc(ufp, etabs)
    x_user0 = jnp.pad(
        jnp.concatenate([newf[:25000], prof[:25000]], axis=0),
        ((0, NPAD - NU), (0, 0)))
    xs0 = _scale_rows_tc(x_user0, dinv)
    x_com0 = jnp.pad(comment_x, ((0, NPAD - NU), (0, 0)))

    seg144_uu = _make_segsum(DU0, CAP_UU)
    seg144_xc = _make_segsum(DU0, CAP_UC)
    seg64_xc = _make_segsum(DC0, CAP_UC, nbuf=2)
    agg_uu0 = seg144_uu(xs0, suu, duu, cuu)
    agg_cu0 = seg64_xc(x_com0, scu, dcu, ccu)
    agg_uc0 = seg144_xc(x_user0, suc, duc, cuc)
    ou0, xs1, oc0 = _hop0_tc(agg_uu0, agg_cu0, agg_uc0, x_user0, x_com0,
                             dinv, iccu, icuc, p)

    seg128_uu = _make_segsum(HID, CAP_UU, nbuf=2)
    seg128_xc = _make_segsum(HID, CAP_UC, nbuf=2)
    agg_uu1 = seg128_uu(xs1, suu, duu, cuu)
    agg_cu1 = seg128_xc(oc0, scu, dcu, ccu)
    node, ps, ps2 = _hop1_tc(agg_uu1, agg_cu1, ou0, dinv, iccu, p)

    coef = _bn_coef_tc(ps, ps2, p["bn_g"], p["bn_b"])
    out = _bn_apply_tc(node, coef)
    return out[:NU]
